# baseline (speedup 1.0000x reference)
import jax
import jax.numpy as jnp
from jax import lax
from jax.experimental import pallas as pl
from jax.experimental.pallas import tpu as pltpu
from jax.experimental.pallas import tpu_sc as plsc

N = 10000
D = 128
DH = D // 2
NPAD = 10240
CH = 128
ROWS_PER_TILE = NPAD // 16
NGRAM = 6
EPS = 1e-5


def _sc_scatter_body(h_hbm, sd_hbm, zeros_hbm, out_hbm,
                     sd_v, rows0_v, rows1_v, rows2_v, rows3_v, agg_sh,
                     sem0, sem1, sem2, sem3):
    c = lax.axis_index("c")
    s = lax.axis_index("s")
    n_chunks = sd_v.shape[1]
    base = s * ROWS_PER_TILE

    bufs = (rows0_v, rows1_v, rows2_v, rows3_v)
    sems = (sem0, sem1, sem2, sem3)

    def _gstart(j, b):
        pltpu.async_copy(h_hbm.at[sd_v.at[0, j]], bufs[b], sems[b])

    def _gwait(b):
        pltpu.make_async_copy(h_hbm.at[sd_v.at[0, 0]], bufs[b], sems[b]).wait()

    def _scat(j, b):
        pltpu.sync_copy(bufs[b], agg_sh.at[sd_v.at[1, j]], add=True)

    zdst = agg_sh.at[pl.ds(base, ROWS_PER_TILE)]
    pltpu.async_copy(zeros_hbm, zdst, sem0)
    pltpu.async_copy(sd_hbm.at[c, s], sd_v, sem1)
    pltpu.make_async_copy(sd_hbm.at[c, s], sd_v, sem1).wait()
    pltpu.make_async_copy(zeros_hbm, zdst, sem0).wait()

    quads = (n_chunks - 3) // 4
    for b in range(3):
        _gstart(b, b)
    plsc.subcore_barrier()

    def _quad(i, carry):
        j = 4 * i
        for b in range(4):
            _gwait(b)
            _scat(j + b, b)
            _gstart(j + b + 3, (b + 3) % 4)
        return carry

    lax.fori_loop(0, quads, _quad, 0)
    j0 = 4 * quads
    for b in range(3):
        _gwait((j0 + b) % 4)
        _scat(j0 + b, (j0 + b) % 4)
    plsc.subcore_barrier()

    pltpu.sync_copy(agg_sh.at[pl.ds(base, ROWS_PER_TILE)],
                    out_hbm.at[c, pl.ds(base, ROWS_PER_TILE)])


def _make_sc_scatter(n_chunks):
    mesh = plsc.VectorSubcoreMesh(core_axis_name="c", subcore_axis_name="s")
    return pl.kernel(
        _sc_scatter_body,
        out_type=jax.ShapeDtypeStruct((2, NPAD, DH), jnp.float32),
        mesh=mesh,
        scratch_types=[
            pltpu.VMEM((2, n_chunks, CH), jnp.int32),
            pltpu.VMEM((CH, DH), jnp.float32),
            pltpu.VMEM((CH, DH), jnp.float32),
            pltpu.VMEM((CH, DH), jnp.float32),
            pltpu.VMEM((CH, DH), jnp.float32),
            pltpu.VMEM_SHARED((NPAD, DH), jnp.float32),
            pltpu.SemaphoreType.DMA,
            pltpu.SemaphoreType.DMA,
            pltpu.SemaphoreType.DMA,
            pltpu.SemaphoreType.DMA,
        ],
        compiler_params=pltpu.CompilerParams(use_tc_tiling_on_sc=False),
        name="sc_edge_scatter_add",
    )


def _tc_dense(agg_ref, h_ref, pool_ref, g, b, mats):
    p0 = agg_ref[0, : N // 2, :]
    p1 = agg_ref[1, : N // 2, :]
    m0 = mats[0]
    he = (jnp.dot(p0[:, :DH], m0[:DH, :], preferred_element_type=jnp.float32)
          + jnp.dot(p1[:, :DH], m0[DH:, :], preferred_element_type=jnp.float32))
    ho = (jnp.dot(p0[:, DH:], m0[:DH, :], preferred_element_type=jnp.float32)
          + jnp.dot(p1[:, DH:], m0[DH:, :], preferred_element_type=jnp.float32))
    for m in mats[1:]:
        he = jnp.dot(he, m, preferred_element_type=jnp.float32)
        ho = jnp.dot(ho, m, preferred_element_type=jnp.float32)
    mean = (jnp.sum(he, axis=0, keepdims=True) +
            jnp.sum(ho, axis=0, keepdims=True)) / N
    var = (jnp.sum(jnp.square(he - mean), axis=0, keepdims=True) +
           jnp.sum(jnp.square(ho - mean), axis=0, keepdims=True)) / N
    inv = lax.rsqrt(var + EPS)
    he = jnp.maximum((he - mean) * inv * g + b, 0.0)
    ho = jnp.maximum((ho - mean) * inv * g + b, 0.0)
    h_ref[0, :, :DH] = he[:, :DH]
    h_ref[0, :, DH:] = ho[:, :DH]
    h_ref[1, :, :DH] = he[:, DH:]
    h_ref[1, :, DH:] = ho[:, DH:]
    pool_ref[...] = (jnp.sum(he, axis=0, keepdims=True) +
                     jnp.sum(ho, axis=0, keepdims=True))


def _tc_first_body(agg_ref, Win_ref, W_ref, g_ref, b_ref, h_ref, pool_ref):
    _tc_dense(agg_ref, h_ref, pool_ref, g_ref[...], b_ref[...],
              (Win_ref[...], W_ref[...]))


def _tc_layer_body(agg_ref, W_ref, g_ref, b_ref, h_ref, pool_ref):
    _tc_dense(agg_ref, h_ref, pool_ref, g_ref[...], b_ref[...],
              (W_ref[...],))


_TC_OUT = [
    jax.ShapeDtypeStruct((2, N // 2, D), jnp.float32),
    jax.ShapeDtypeStruct((1, D), jnp.float32),
]

_tc_first = pl.pallas_call(_tc_first_body, out_shape=_TC_OUT)
_tc_layer = pl.pallas_call(_tc_layer_body, out_shape=_TC_OUT)


def _head_body(pool_ref, w_ref, W1_ref, b1_ref, W2_ref, b2_ref, out_ref):
    w = jax.nn.softmax(w_ref[...], axis=-1)
    comb = jnp.dot(w, pool_ref[...], preferred_element_type=jnp.float32)
    o = jnp.dot(comb, W1_ref[...], preferred_element_type=jnp.float32)
    o = o + b1_ref[...]
    o = jnp.where(o > 0, o, 0.01 * o)
    o = jnp.dot(o, W2_ref[...], preferred_element_type=jnp.float32)
    o = o + b2_ref[...]
    out_ref[...] = jax.nn.sigmoid(o)


_head = pl.pallas_call(
    _head_body, out_shape=jax.ShapeDtypeStruct((1, 1), jnp.float32))


def _prep_edges(edge_index):
    src = edge_index[0].astype(jnp.int32)
    dst = edge_index[1].astype(jnp.int32)
    e = src.shape[0]
    n_chunks = -(-e // (16 * CH))
    while n_chunks % 4 != 3:
        n_chunks += 1
    e_pad = 16 * n_chunks * CH
    pad = e_pad - e
    ar = jnp.arange(pad, dtype=jnp.int32)
    src_p = jnp.concatenate([src, ar % N]).reshape(1, 16, 1, n_chunks, CH)
    dst_p = jnp.concatenate([dst, N + ar % (NPAD - N)]).reshape(
        1, 16, 1, n_chunks, CH)
    dst_2 = jnp.concatenate([dst_p, dst_p], axis=0)
    src_2 = jnp.concatenate([src_p, src_p + N], axis=0)
    src_x = jnp.concatenate([2 * src_p, 2 * src_p + 1], axis=0)
    sd_p = jnp.concatenate([src_2, dst_2], axis=2)
    sd_x = jnp.concatenate([src_x, dst_2], axis=2)
    return sd_p, sd_x, n_chunks


def kernel(x, edge_index, W_in, W_conv1, gamma1, beta1, W_conv2, gamma2,
           beta2, ngram_weights, W_lin1, b_lin1, W_lin2, b_lin2):
    sd_p, sd_x, n_chunks = _prep_edges(edge_index)
    zeros_stripe = jnp.zeros((ROWS_PER_TILE, DH), jnp.float32)
    sc_scatter = _make_sc_scatter(n_chunks)

    g1 = gamma1.reshape(1, D)
    b1 = beta1.reshape(1, D)
    g2 = gamma2.reshape(1, D)
    b2 = beta2.reshape(1, D)

    hflat = x.reshape(2 * N, DH)
    pools = []
    for g in range(NGRAM):
        agg = sc_scatter(hflat, sd_x if g == 0 else sd_p, zeros_stripe)
        agg128 = agg.reshape(2, NPAD // 2, D)
        if g == 0:
            h2, _ = _tc_first(agg128, W_in, W_conv1, g1, b1)
        else:
            h2, _ = _tc_layer(agg128, W_conv1, g1, b1)
        agg = sc_scatter(h2.reshape(2 * N, DH), sd_p, zeros_stripe)
        agg128 = agg.reshape(2, NPAD // 2, D)
        h2, pool = _tc_layer(agg128, W_conv2, g2, b2)
        hflat = h2.reshape(2 * N, DH)
        pools.append(pool)

    pools8 = jnp.concatenate(pools + [jnp.zeros((2, D), jnp.float32)], axis=0)
    w8 = jnp.concatenate(
        [ngram_weights, jnp.full((2,), -1e30, jnp.float32)]).reshape(1, 8)
    return _head(pools8, w8, W_lin1, b_lin1.reshape(1, -1),
                 W_lin2, b_lin2.reshape(1, -1))


# speedup vs baseline: 1.0135x; 1.0135x over previous
import jax
import jax.numpy as jnp
from jax import lax
from jax.experimental import pallas as pl
from jax.experimental.pallas import tpu as pltpu
from jax.experimental.pallas import tpu_sc as plsc

N = 10000
D = 128
DH = D // 2
NPAD = 10240
CH = 128
ROWS_PER_TILE = NPAD // 16
NGRAM = 6
EPS = 1e-5


def _sc_scatter_body(h_hbm, sd_hbm, zeros_hbm, out_hbm,
                     sd_v, rows0_v, rows1_v, rows2_v, rows3_v, agg_sh,
                     sem0, sem1, sem2, sem3):
    c = lax.axis_index("c")
    s = lax.axis_index("s")
    n_chunks = sd_v.shape[1]
    base = s * ROWS_PER_TILE

    bufs = (rows0_v, rows1_v, rows2_v, rows3_v)
    sems = (sem0, sem1, sem2, sem3)

    def _gstart(j, b):
        pltpu.async_copy(h_hbm.at[sd_v.at[0, j]], bufs[b], sems[b])

    def _gwait(b):
        pltpu.make_async_copy(h_hbm.at[sd_v.at[0, 0]], bufs[b], sems[b]).wait()

    def _scat(j, b):
        pltpu.sync_copy(bufs[b], agg_sh.at[sd_v.at[1, j]], add=True)

    zdst = agg_sh.at[pl.ds(base, ROWS_PER_TILE)]
    pltpu.async_copy(zeros_hbm, zdst, sem0)
    pltpu.async_copy(sd_hbm.at[c, s], sd_v, sem1)
    pltpu.make_async_copy(sd_hbm.at[c, s], sd_v, sem1).wait()
    pltpu.make_async_copy(zeros_hbm, zdst, sem0).wait()

    quads = (n_chunks - 3) // 4
    for b in range(3):
        _gstart(b, b)
    plsc.subcore_barrier()

    def _quad(i, carry):
        j = 4 * i
        for b in range(4):
            _gwait(b)
            _scat(j + b, b)
            _gstart(j + b + 3, (b + 3) % 4)
        return carry

    lax.fori_loop(0, quads, _quad, 0)
    j0 = 4 * quads
    for b in range(3):
        _gwait((j0 + b) % 4)
        _scat(j0 + b, (j0 + b) % 4)
    plsc.subcore_barrier()

    pltpu.sync_copy(agg_sh.at[pl.ds(base, ROWS_PER_TILE)],
                    out_hbm.at[c, pl.ds(base, ROWS_PER_TILE)])


def _make_sc_scatter(n_chunks):
    mesh = plsc.VectorSubcoreMesh(core_axis_name="c", subcore_axis_name="s")
    return pl.kernel(
        _sc_scatter_body,
        out_type=jax.ShapeDtypeStruct((2, NPAD, DH), jnp.float32),
        mesh=mesh,
        scratch_types=[
            pltpu.VMEM((2, n_chunks, CH), jnp.int32),
            pltpu.VMEM((CH, DH), jnp.float32),
            pltpu.VMEM((CH, DH), jnp.float32),
            pltpu.VMEM((CH, DH), jnp.float32),
            pltpu.VMEM((CH, DH), jnp.float32),
            pltpu.VMEM_SHARED((NPAD, DH), jnp.float32),
            pltpu.SemaphoreType.DMA,
            pltpu.SemaphoreType.DMA,
            pltpu.SemaphoreType.DMA,
            pltpu.SemaphoreType.DMA,
        ],
        compiler_params=pltpu.CompilerParams(use_tc_tiling_on_sc=False),
        name="sc_edge_scatter_add",
    )


def _tc_dense(agg_ref, h_ref, pool_ref, g, b, mats):
    p0 = agg_ref[0, : N // 2, :]
    p1 = agg_ref[1, : N // 2, :]
    he = jnp.concatenate([p0[:, :DH], p1[:, :DH]], axis=1)
    ho = jnp.concatenate([p0[:, DH:], p1[:, DH:]], axis=1)
    for m in mats:
        he = jnp.dot(he, m, preferred_element_type=jnp.float32)
        ho = jnp.dot(ho, m, preferred_element_type=jnp.float32)
    mean = (jnp.sum(he, axis=0, keepdims=True) +
            jnp.sum(ho, axis=0, keepdims=True)) / N
    var = (jnp.sum(jnp.square(he - mean), axis=0, keepdims=True) +
           jnp.sum(jnp.square(ho - mean), axis=0, keepdims=True)) / N
    inv = lax.rsqrt(var + EPS)
    he = jnp.maximum((he - mean) * inv * g + b, 0.0)
    ho = jnp.maximum((ho - mean) * inv * g + b, 0.0)
    h_ref[0, :, :] = jnp.concatenate([he[:, :DH], ho[:, :DH]], axis=1)
    h_ref[1, :, :] = jnp.concatenate([he[:, DH:], ho[:, DH:]], axis=1)
    pool_ref[...] = (jnp.sum(he, axis=0, keepdims=True) +
                     jnp.sum(ho, axis=0, keepdims=True))


def _tc_first_body(agg_ref, Win_ref, W_ref, g_ref, b_ref, h_ref, pool_ref):
    _tc_dense(agg_ref, h_ref, pool_ref, g_ref[...], b_ref[...],
              (Win_ref[...], W_ref[...]))


def _tc_layer_body(agg_ref, W_ref, g_ref, b_ref, h_ref, pool_ref):
    _tc_dense(agg_ref, h_ref, pool_ref, g_ref[...], b_ref[...],
              (W_ref[...],))


_TC_OUT = [
    jax.ShapeDtypeStruct((2, N // 2, D), jnp.float32),
    jax.ShapeDtypeStruct((1, D), jnp.float32),
]

_tc_first = pl.pallas_call(_tc_first_body, out_shape=_TC_OUT)
_tc_layer = pl.pallas_call(_tc_layer_body, out_shape=_TC_OUT)


def _head_body(pool_ref, w_ref, W1_ref, b1_ref, W2_ref, b2_ref, out_ref):
    w = jax.nn.softmax(w_ref[...], axis=-1)
    comb = jnp.dot(w, pool_ref[...], preferred_element_type=jnp.float32)
    o = jnp.dot(comb, W1_ref[...], preferred_element_type=jnp.float32)
    o = o + b1_ref[...]
    o = jnp.where(o > 0, o, 0.01 * o)
    o = jnp.dot(o, W2_ref[...], preferred_element_type=jnp.float32)
    o = o + b2_ref[...]
    out_ref[...] = jax.nn.sigmoid(o)


_head = pl.pallas_call(
    _head_body, out_shape=jax.ShapeDtypeStruct((1, 1), jnp.float32))


def _prep_edges(edge_index):
    src = edge_index[0].astype(jnp.int32)
    dst = edge_index[1].astype(jnp.int32)
    e = src.shape[0]
    n_chunks = -(-e // (16 * CH))
    while n_chunks % 4 != 3:
        n_chunks += 1
    e_pad = 16 * n_chunks * CH
    pad = e_pad - e
    ar = jnp.arange(pad, dtype=jnp.int32)
    src_p = jnp.concatenate([src, ar % N]).reshape(1, 16, 1, n_chunks, CH)
    dst_p = jnp.concatenate([dst, N + ar % (NPAD - N)]).reshape(
        1, 16, 1, n_chunks, CH)
    dst_2 = jnp.concatenate([dst_p, dst_p], axis=0)
    src_2 = jnp.concatenate([src_p, src_p + N], axis=0)
    src_x = jnp.concatenate([2 * src_p, 2 * src_p + 1], axis=0)
    sd_p = jnp.concatenate([src_2, dst_2], axis=2)
    sd_x = jnp.concatenate([src_x, dst_2], axis=2)
    return sd_p, sd_x, n_chunks


def kernel(x, edge_index, W_in, W_conv1, gamma1, beta1, W_conv2, gamma2,
           beta2, ngram_weights, W_lin1, b_lin1, W_lin2, b_lin2):
    sd_p, sd_x, n_chunks = _prep_edges(edge_index)
    zeros_stripe = jnp.zeros((ROWS_PER_TILE, DH), jnp.float32)
    sc_scatter = _make_sc_scatter(n_chunks)

    g1 = gamma1.reshape(1, D)
    b1 = beta1.reshape(1, D)
    g2 = gamma2.reshape(1, D)
    b2 = beta2.reshape(1, D)

    hflat = x.reshape(2 * N, DH)
    pools = []
    for g in range(NGRAM):
        agg = sc_scatter(hflat, sd_x if g == 0 else sd_p, zeros_stripe)
        agg128 = agg.reshape(2, NPAD // 2, D)
        if g == 0:
            h2, _ = _tc_first(agg128, W_in, W_conv1, g1, b1)
        else:
            h2, _ = _tc_layer(agg128, W_conv1, g1, b1)
        agg = sc_scatter(h2.reshape(2 * N, DH), sd_p, zeros_stripe)
        agg128 = agg.reshape(2, NPAD // 2, D)
        h2, pool = _tc_layer(agg128, W_conv2, g2, b2)
        hflat = h2.reshape(2 * N, DH)
        pools.append(pool)

    pools8 = jnp.concatenate(pools + [jnp.zeros((2, D), jnp.float32)], axis=0)
    w8 = jnp.concatenate(
        [ngram_weights, jnp.full((2,), -1e30, jnp.float32)]).reshape(1, 8)
    return _head(pools8, w8, W_lin1, b_lin1.reshape(1, -1),
                 W_lin2, b_lin2.reshape(1, -1))
